# baseline (speedup 1.0000x reference)
import functools

import jax
import jax.numpy as jnp
from jax import lax
from jax.experimental import pallas as pl
from jax.experimental.pallas import tpu as pltpu
from jax.experimental.pallas import tpu_sc as plsc

MAXLEN = 200
EMBED = 64
NC = 2
NS = 16
NW = NC * NS
CHUNK = 100
LANES = 16


def _body(idx_hbm, tok_hbm, pos_hbm, out_hbm, idx_v, pos_v, rows_v, gsem):
    cid = lax.axis_index("c")
    sid = lax.axis_index("s")
    wid = sid * NC + cid

    pltpu.sync_copy(idx_hbm.at[wid], idx_v)
    pltpu.sync_copy(pos_hbm, pos_v)

    nchunk = idx_v.shape[0]

    def chunk_body(c, _):
        pltpu.async_copy(tok_hbm.at[idx_v.at[c]], rows_v, gsem).wait()
        po = lax.rem(c, 2) * CHUNK

        def row_body(r, _):
            for j in range(EMBED // LANES):
                s = pl.ds(j * LANES, LANES)
                rows_v[r, s] = rows_v[r, s] + pos_v[po + r, s]
            return 0

        lax.fori_loop(0, CHUNK, row_body, 0)
        pltpu.sync_copy(rows_v, out_hbm.at[wid * nchunk + c])
        return 0

    lax.fori_loop(0, nchunk, chunk_body, 0)


def kernel(x, tok_table, pos_table):
    B, L = x.shape
    total = B * L
    assert total % (NW * CHUNK) == 0
    nchunk = total // (NW * CHUNK)
    idx = x.reshape(NW, nchunk, CHUNK)

    mesh = plsc.VectorSubcoreMesh(core_axis_name="c", subcore_axis_name="s")
    run = functools.partial(
        pl.kernel,
        mesh=mesh,
        compiler_params=pltpu.CompilerParams(use_tc_tiling_on_sc=False),
        out_type=jax.ShapeDtypeStruct((NW * nchunk, CHUNK, EMBED), jnp.float32),
        scratch_types=[
            pltpu.VMEM((nchunk, CHUNK), jnp.int32),
            pltpu.VMEM((MAXLEN, EMBED), jnp.float32),
            pltpu.VMEM((CHUNK, EMBED), jnp.float32),
            pltpu.SemaphoreType.DMA,
        ],
    )(_body)
    out = run(idx, tok_table, pos_table)
    return out.reshape(B, L, EMBED)


# speedup vs baseline: 1.4945x; 1.4945x over previous
import functools

import jax
import jax.numpy as jnp
from jax import lax
from jax.experimental import pallas as pl
from jax.experimental.pallas import tpu as pltpu
from jax.experimental.pallas import tpu_sc as plsc

MAXLEN = 200
EMBED = 64
NC = 2
NS = 16
NW = NC * NS
CHUNK = 100
LANES = 16
NBUF = 4
AHEAD = 2


def _body(idx_hbm, tok_hbm, pos_hbm, out_hbm, idx_v, pos_v, rows_v,
          g0, g1, g2, g3, o0, o1, o2, o3):
    gsems = (g0, g1, g2, g3)
    osems = (o0, o1, o2, o3)
    cid = lax.axis_index("c")
    sid = lax.axis_index("s")
    wid = sid * NC + cid

    pltpu.sync_copy(idx_hbm.at[wid], idx_v)
    pltpu.sync_copy(pos_hbm, pos_v)

    nchunk = idx_v.shape[0]
    nblock = nchunk // NBUF

    def gather_start(c, b):
        pltpu.async_copy(tok_hbm.at[idx_v.at[c]], rows_v.at[b], gsems[b])

    def gather_wait(c, b):
        pltpu.make_async_copy(
            tok_hbm.at[idx_v.at[c]], rows_v.at[b], gsems[b]).wait()

    def write_start(c, b):
        pltpu.async_copy(rows_v.at[b], out_hbm.at[wid * nchunk + c], osems[b])

    def write_wait(c, b):
        pltpu.make_async_copy(
            rows_v.at[b], out_hbm.at[wid * nchunk + c], osems[b]).wait()

    def compute(c, b):
        po = lax.rem(c, 2) * CHUNK

        def row2(r2, _):
            for dr in range(2):
                r = r2 * 2 + dr
                for j in range(EMBED // LANES):
                    s = pl.ds(j * LANES, LANES)
                    rows_v[b, r, s] = rows_v[b, r, s] + pos_v[po + r, s]
            return 0

        lax.fori_loop(0, CHUNK // 2, row2, 0)

    def step(c, b):
        gather_wait(c, b)
        compute(c, b)
        write_start(c, b)
        cf = c + AHEAD
        bf = (b + AHEAD) % NBUF

        def prefetch(cf=cf, bf=bf):
            write_wait(cf - NBUF, bf)
            gather_start(cf, bf)

        return prefetch

    for c in range(AHEAD):
        gather_start(c, c % NBUF)

    for b in range(NBUF):
        gather_wait(b, b)
        compute(b, b)
        write_start(b, b)
        cf, bf = b + AHEAD, (b + AHEAD) % NBUF
        if cf - NBUF >= 0:
            write_wait(cf - NBUF, bf)
        gather_start(cf, bf)

    def block(k, _):
        c0 = k * NBUF
        for b in range(NBUF):
            step(c0 + b, b)()
        return 0

    lax.fori_loop(1, nblock - 1, block, 0)

    c0 = (nblock - 1) * NBUF
    for b in range(NBUF):
        c = c0 + b
        gather_wait(c, b)
        compute(c, b)
        write_start(c, b)
        cf, bf = c + AHEAD, (b + AHEAD) % NBUF
        if cf < nchunk:
            write_wait(cf - NBUF, bf)
            gather_start(cf, bf)

    for c in range(nchunk - NBUF, nchunk):
        write_wait(c, c % NBUF)


def kernel(x, tok_table, pos_table):
    B, L = x.shape
    total = B * L
    assert total % (NW * CHUNK) == 0
    nchunk = total // (NW * CHUNK)
    idx = x.reshape(NW, nchunk, CHUNK)

    mesh = plsc.VectorSubcoreMesh(core_axis_name="c", subcore_axis_name="s")
    run = functools.partial(
        pl.kernel,
        mesh=mesh,
        compiler_params=pltpu.CompilerParams(use_tc_tiling_on_sc=False),
        out_type=jax.ShapeDtypeStruct((NW * nchunk, CHUNK, EMBED), jnp.float32),
        scratch_types=[
            pltpu.VMEM((nchunk, CHUNK), jnp.int32),
            pltpu.VMEM((MAXLEN, EMBED), jnp.float32),
            pltpu.VMEM((NBUF, CHUNK, EMBED), jnp.float32),
        ] + [pltpu.SemaphoreType.DMA] * (2 * NBUF),
    )(_body)
    out = run(idx, tok_table, pos_table)
    return out.reshape(B, L, EMBED)


# speedup vs baseline: 1.5527x; 1.0389x over previous
import functools

import jax
import jax.numpy as jnp
from jax import lax
from jax.experimental import pallas as pl
from jax.experimental.pallas import tpu as pltpu
from jax.experimental.pallas import tpu_sc as plsc

MAXLEN = 200
EMBED = 64
NC = 2
NS = 16
NW = NC * NS
LANES = 16
NBUF = 4
AHEAD = 2


def _body(xt_hbm, tok_hbm, pos_hbm, out_hbm, idx_v, pos_v, rows_v,
          g0, g1, g2, g3, o0, o1, o2, o3):
    gsems = (g0, g1, g2, g3)
    osems = (o0, o1, o2, o3)
    cid = lax.axis_index("c")
    sid = lax.axis_index("s")
    wid = sid * NC + cid
    nb = xt_hbm.shape[1] // NW
    b0 = wid * nb
    nchunk = xt_hbm.shape[0]
    nblock = nchunk // NBUF

    pltpu.sync_copy(xt_hbm.at[:, pl.ds(b0, nb)], idx_v)
    pltpu.sync_copy(pos_hbm, pos_v)

    def gather_start(l, b):
        pltpu.async_copy(tok_hbm.at[idx_v.at[l]], rows_v.at[b], gsems[b])

    def gather_wait(l, b):
        pltpu.make_async_copy(
            tok_hbm.at[idx_v.at[l]], rows_v.at[b], gsems[b]).wait()

    def write_start(l, b):
        pltpu.async_copy(rows_v.at[b], out_hbm.at[pl.ds(b0, nb), l], osems[b])

    def write_wait(l, b):
        pltpu.make_async_copy(
            rows_v.at[b], out_hbm.at[pl.ds(b0, nb), l], osems[b]).wait()

    def compute(l, b):
        prow = [pos_v[l, pl.ds(j * LANES, LANES)] for j in range(EMBED // LANES)]

        def row2(r2, _):
            for dr in range(2):
                r = r2 * 2 + dr
                for j in range(EMBED // LANES):
                    s = pl.ds(j * LANES, LANES)
                    rows_v[b, r, s] = rows_v[b, r, s] + prow[j]
            return 0

        lax.fori_loop(0, nb // 2, row2, 0)

    for l in range(AHEAD):
        gather_start(l, l % NBUF)

    for b in range(NBUF):
        gather_wait(b, b)
        compute(b, b)
        write_start(b, b)
        lf, bf = b + AHEAD, (b + AHEAD) % NBUF
        if lf - NBUF >= 0:
            write_wait(lf - NBUF, bf)
        gather_start(lf, bf)

    def block(k, _):
        l0 = k * NBUF
        for b in range(NBUF):
            l = l0 + b
            gather_wait(l, b)
            compute(l, b)
            write_start(l, b)
            lf, bf = l + AHEAD, (b + AHEAD) % NBUF
            write_wait(lf - NBUF, bf)
            gather_start(lf, bf)
        return 0

    lax.fori_loop(1, nblock - 1, block, 0)

    l0 = (nblock - 1) * NBUF
    for b in range(NBUF):
        l = l0 + b
        gather_wait(l, b)
        compute(l, b)
        write_start(l, b)
        lf, bf = l + AHEAD, (b + AHEAD) % NBUF
        if lf < nchunk:
            write_wait(lf - NBUF, bf)
            gather_start(lf, bf)

    for l in range(nchunk - NBUF, nchunk):
        write_wait(l, l % NBUF)


def kernel(x, tok_table, pos_table):
    B, L = x.shape
    assert B % NW == 0 and L % NBUF == 0
    nb = B // NW

    mesh = plsc.VectorSubcoreMesh(core_axis_name="c", subcore_axis_name="s")
    run = functools.partial(
        pl.kernel,
        mesh=mesh,
        compiler_params=pltpu.CompilerParams(use_tc_tiling_on_sc=False),
        out_type=jax.ShapeDtypeStruct((B, L, EMBED), jnp.float32),
        scratch_types=[
            pltpu.VMEM((L, nb), jnp.int32),
            pltpu.VMEM((L, EMBED), jnp.float32),
            pltpu.VMEM((NBUF, nb, EMBED), jnp.float32),
        ] + [pltpu.SemaphoreType.DMA] * (2 * NBUF),
    )(_body)
    return run(x.T, tok_table, pos_table)
